# baseline (speedup 1.0000x reference)
import functools

import jax
import jax.numpy as jnp
from jax import lax
from jax.experimental import pallas as pl
from jax.experimental.pallas import tpu as pltpu
from jax.experimental.pallas import tpu_sc as plsc

T = 2048
H = 768
F = 1280
E = 12
GATE = 4
BLK = 128
NB = T * 2 // BLK + E
PAD = NB * BLK
NTILES = 32
RPT = PAD // NTILES
TPT = T // NTILES
CH = 256
L = 16


def _router_body(h_ref, gwt_ref, posa_ref, posb_ref, wa_ref, wb_ref, be_ref):
    h = h_ref[...]
    logits = jnp.dot(h, gwt_ref[...], preferred_element_type=jnp.float32)
    iota_e = lax.broadcasted_iota(jnp.int32, (T, E), 1)
    m1 = jnp.max(logits, axis=1, keepdims=True)
    a1 = jnp.min(jnp.where(logits >= m1, iota_e, E), axis=1, keepdims=True)
    oh1 = iota_e == a1
    l2 = jnp.where(oh1, jnp.float32(-1e30), logits)
    m2 = jnp.max(l2, axis=1, keepdims=True)
    a2 = jnp.min(jnp.where(l2 >= m2, iota_e, E), axis=1, keepdims=True)
    oh2 = iota_e == a2
    wa = 1.0 / (1.0 + jnp.exp(m2 - m1))
    wa_ref[...] = wa
    wb_ref[...] = 1.0 - wa

    ohc = oh1.astype(jnp.float32) + oh2.astype(jnp.float32)
    r = lax.broadcasted_iota(jnp.int32, (CH, CH), 0)
    c = lax.broadcasted_iota(jnp.int32, (CH, CH), 1)
    stril = (c < r).astype(jnp.float32)
    parts = []
    run = jnp.zeros((1, E), jnp.float32)
    for k in range(T // CH):
        blk = lax.slice(ohc, (k * CH, 0), ((k + 1) * CH, E))
        parts.append(jnp.dot(stril, blk, preferred_element_type=jnp.float32) + run)
        run = run + jnp.sum(blk, axis=0, keepdims=True)
    excl = jnp.concatenate(parts, axis=0)
    counts = run.astype(jnp.int32)
    pc = ((counts + (BLK - 1)) // BLK) * BLK
    rr = lax.broadcasted_iota(jnp.int32, (E, E), 0)
    cc = lax.broadcasted_iota(jnp.int32, (E, E), 1)
    striu = (rr < cc).astype(jnp.float32)
    po = jnp.dot(pc.astype(jnp.float32), striu,
                 preferred_element_type=jnp.float32)
    posf = excl + po
    posa_ref[...] = jnp.sum(jnp.where(oh1, posf, 0.0), axis=1,
                            keepdims=True).astype(jnp.int32)
    posb_ref[...] = jnp.sum(jnp.where(oh2, posf, 0.0), axis=1,
                            keepdims=True).astype(jnp.int32)
    total = jnp.sum(pc)
    sb = jnp.minimum(lax.broadcasted_iota(jnp.int32, (NB, 1), 0) * BLK,
                     total - 1)
    be_ref[...] = jnp.sum((po.astype(jnp.int32) <= sb).astype(jnp.int32),
                          axis=1, keepdims=True) - 1


_router = pl.pallas_call(
    _router_body,
    out_shape=(
        jax.ShapeDtypeStruct((T, 1), jnp.int32),
        jax.ShapeDtypeStruct((T, 1), jnp.int32),
        jax.ShapeDtypeStruct((T, 1), jnp.float32),
        jax.ShapeDtypeStruct((T, 1), jnp.float32),
        jax.ShapeDtypeStruct((NB, 1), jnp.int32),
    ),
)


@functools.cache
def _sc_mesh():
    return plsc.VectorSubcoreMesh(core_axis_name="c", subcore_axis_name="s")


@functools.cache
def _gather_kernel():
    return pl.kernel(
        _gather_body,
        mesh=_sc_mesh(),
        out_type=(
            jax.ShapeDtypeStruct((PAD, H), jnp.float32),
            jax.ShapeDtypeStruct((PAD,), jnp.float32),
        ),
        scratch_types=[
            pltpu.VMEM((2 * T,), jnp.int32),
            pltpu.VMEM((2 * T,), jnp.float32),
            pltpu.VMEM((PAD,), jnp.int32),
            pltpu.VMEM((PAD,), jnp.float32),
            pltpu.VMEM((RPT // 2, H), jnp.float32),
            pltpu.SemaphoreType.DMA,
        ],
        compiler_params=pltpu.CompilerParams(needs_layout_passes=False),
    )


def _gather_body(h_hbm, pos_hbm, wn_hbm, hs_hbm, rw_hbm,
                 pos_v, wn_v, tok_v, rw_v, buf_v, sem):
    wid = lax.axis_index("s") * 2 + lax.axis_index("c")
    pltpu.sync_copy(pos_hbm, pos_v)
    pltpu.sync_copy(wn_hbm, wn_v)
    iota_init = lax.iota(jnp.int32, L)
    zf = jnp.zeros((L,), jnp.float32)

    def init_body(i, carry):
        sl = pl.ds(i * L, L)
        tok_v[sl] = (iota_init + i * L) & (T - 1)
        rw_v[sl] = zf
        return carry

    lax.fori_loop(0, PAD // L, init_body, 0)
    iota16 = lax.iota(jnp.int32, L)

    def scat_body(i, carry):
        sl = pl.ds(i * L, L)
        idx = pos_v[sl]
        plsc.store_scatter(tok_v, [idx], (iota16 + i * L) & (T - 1))
        plsc.store_scatter(rw_v, [idx], wn_v[sl])
        return carry

    lax.fori_loop(0, 2 * T // L, scat_body, 0)
    base = wid * RPT
    pltpu.sync_copy(rw_v.at[pl.ds(base, RPT)], rw_hbm.at[pl.ds(base, RPT)])
    for half in range(2):
        st = base + half * (RPT // 2)
        pltpu.async_copy(h_hbm.at[tok_v.at[pl.ds(st, RPT // 2)]],
                         buf_v, sem).wait()
        pltpu.sync_copy(buf_v, hs_hbm.at[pl.ds(st, RPT // 2)])


def _ffn_body(be_ref, h_ref, rw_ref, w1_ref, w3_ref, w2_ref, y_ref):
    del be_ref
    h = h_ref[...].astype(jnp.bfloat16)
    w1 = w1_ref[0].astype(jnp.bfloat16)
    w3 = w3_ref[0].astype(jnp.bfloat16)
    w2 = w2_ref[0].astype(jnp.bfloat16)
    nt = (((1,), (1,)), ((), ()))
    f1 = lax.dot_general(h, w1, nt, preferred_element_type=jnp.float32)
    f3 = lax.dot_general(h, w3, nt, preferred_element_type=jnp.float32)
    z = f1 * f3
    x = z / (1.0 + jnp.exp(-z))
    y = lax.dot_general(x.astype(jnp.bfloat16), w2, nt,
                        preferred_element_type=jnp.float32)
    y_ref[...] = y * rw_ref[...]


_ffn = pl.pallas_call(
    _ffn_body,
    grid_spec=pltpu.PrefetchScalarGridSpec(
        num_scalar_prefetch=1,
        grid=(NB,),
        in_specs=[
            pl.BlockSpec((BLK, H), lambda i, be: (i, 0)),
            pl.BlockSpec((BLK, 1), lambda i, be: (i, 0)),
            pl.BlockSpec((1, F, H), lambda i, be: (be[i], 0, 0)),
            pl.BlockSpec((1, F, H), lambda i, be: (be[i], 0, 0)),
            pl.BlockSpec((1, H, F), lambda i, be: (be[i], 0, 0)),
        ],
        out_specs=pl.BlockSpec((BLK, H), lambda i, be: (i, 0)),
    ),
    out_shape=jax.ShapeDtypeStruct((PAD, H), jnp.float32),
)


@functools.cache
def _combine_kernel():
    return pl.kernel(
        _combine_body,
        mesh=_sc_mesh(),
        out_type=jax.ShapeDtypeStruct((T, H), jnp.float32),
        scratch_types=[
            pltpu.VMEM((TPT,), jnp.int32),
            pltpu.VMEM((TPT,), jnp.int32),
            pltpu.VMEM((TPT, H), jnp.float32),
            pltpu.VMEM((TPT, H), jnp.float32),
            pltpu.SemaphoreType.DMA,
        ],
        compiler_params=pltpu.CompilerParams(needs_layout_passes=False),
    )


def _combine_body(y_hbm, pos_hbm, out_hbm, ia_v, ib_v, bufa_v, bufb_v, sem):
    wid = lax.axis_index("s") * 2 + lax.axis_index("c")
    bt = wid * TPT
    pltpu.sync_copy(pos_hbm.at[pl.ds(bt, TPT)], ia_v)
    pltpu.sync_copy(pos_hbm.at[pl.ds(T + bt, TPT)], ib_v)
    pltpu.async_copy(y_hbm.at[ia_v], bufa_v, sem).wait()
    pltpu.async_copy(y_hbm.at[ib_v], bufb_v, sem).wait()

    def tok_body(t, carry):
        for j in range(H // L):
            sl = pl.ds(j * L, L)
            bufa_v[t, sl] = bufa_v[t, sl] + bufb_v[t, sl]
        return carry

    lax.fori_loop(0, TPT, tok_body, 0)
    pltpu.sync_copy(bufa_v, out_hbm.at[pl.ds(bt, TPT)])


def kernel(hidden_states, gate_w, w1, w3, w2):
    h = hidden_states.reshape(T, H)
    gwt = gate_w[GATE].T
    posa, posb, wa, wb, be = _router(h, gwt)
    pos_flat = jnp.concatenate([posa[:, 0], posb[:, 0]])
    wn_flat = jnp.concatenate([wa[:, 0], wb[:, 0]])
    hs, rw = _gather_kernel()(h, pos_flat, wn_flat)
    y = _ffn(be[:, 0], hs, rw.reshape(PAD, 1), w1, w3, w2)
    out = _combine_kernel()(y, pos_flat)
    return out.reshape(1, T, H)


# speedup vs baseline: 1.2220x; 1.2220x over previous
import functools

import jax
import jax.numpy as jnp
from jax import lax
from jax.experimental import pallas as pl
from jax.experimental.pallas import tpu as pltpu
from jax.experimental.pallas import tpu_sc as plsc

T = 2048
H = 768
F = 1280
E = 12
GATE = 4
BLK = 256
NB = T * 2 // BLK + E
PAD = NB * BLK
NTILES = 32
RPT = PAD // NTILES
TPT = T // NTILES
CH = 256
L = 16


def _router_body(h_ref, gwt_ref, posa_ref, posb_ref, wa_ref, wb_ref, be_ref):
    h = h_ref[...]
    logits = jnp.dot(h, gwt_ref[...], preferred_element_type=jnp.float32)
    iota_e = lax.broadcasted_iota(jnp.int32, (T, E), 1)
    m1 = jnp.max(logits, axis=1, keepdims=True)
    a1 = jnp.min(jnp.where(logits >= m1, iota_e, E), axis=1, keepdims=True)
    oh1 = iota_e == a1
    l2 = jnp.where(oh1, jnp.float32(-1e30), logits)
    m2 = jnp.max(l2, axis=1, keepdims=True)
    a2 = jnp.min(jnp.where(l2 >= m2, iota_e, E), axis=1, keepdims=True)
    oh2 = iota_e == a2
    wa = 1.0 / (1.0 + jnp.exp(m2 - m1))
    wa_ref[...] = wa
    wb_ref[...] = 1.0 - wa

    ohc = oh1.astype(jnp.float32) + oh2.astype(jnp.float32)
    r = lax.broadcasted_iota(jnp.int32, (CH, CH), 0)
    c = lax.broadcasted_iota(jnp.int32, (CH, CH), 1)
    stril = (c < r).astype(jnp.float32)
    parts = []
    run = jnp.zeros((1, E), jnp.float32)
    for k in range(T // CH):
        blk = lax.slice(ohc, (k * CH, 0), ((k + 1) * CH, E))
        parts.append(jnp.dot(stril, blk, preferred_element_type=jnp.float32) + run)
        run = run + jnp.sum(blk, axis=0, keepdims=True)
    excl = jnp.concatenate(parts, axis=0)
    counts = run.astype(jnp.int32)
    pc = ((counts + (BLK - 1)) // BLK) * BLK
    rr = lax.broadcasted_iota(jnp.int32, (E, E), 0)
    cc = lax.broadcasted_iota(jnp.int32, (E, E), 1)
    striu = (rr < cc).astype(jnp.float32)
    po = jnp.dot(pc.astype(jnp.float32), striu,
                 preferred_element_type=jnp.float32)
    posf = excl + po
    posa_ref[...] = jnp.sum(jnp.where(oh1, posf, 0.0), axis=1,
                            keepdims=True).astype(jnp.int32)
    posb_ref[...] = jnp.sum(jnp.where(oh2, posf, 0.0), axis=1,
                            keepdims=True).astype(jnp.int32)
    total = jnp.sum(pc)
    sb = jnp.minimum(lax.broadcasted_iota(jnp.int32, (NB, 1), 0) * BLK,
                     total - 1)
    be_ref[...] = jnp.sum((po.astype(jnp.int32) <= sb).astype(jnp.int32),
                          axis=1, keepdims=True) - 1


_router = pl.pallas_call(
    _router_body,
    out_shape=(
        jax.ShapeDtypeStruct((T, 1), jnp.int32),
        jax.ShapeDtypeStruct((T, 1), jnp.int32),
        jax.ShapeDtypeStruct((T, 1), jnp.float32),
        jax.ShapeDtypeStruct((T, 1), jnp.float32),
        jax.ShapeDtypeStruct((NB, 1), jnp.int32),
    ),
)


@functools.cache
def _sc_mesh():
    return plsc.VectorSubcoreMesh(core_axis_name="c", subcore_axis_name="s")


@functools.cache
def _gather_kernel():
    return pl.kernel(
        _gather_body,
        mesh=_sc_mesh(),
        out_type=(
            jax.ShapeDtypeStruct((PAD, H), jnp.float32),
            jax.ShapeDtypeStruct((PAD,), jnp.float32),
        ),
        scratch_types=[
            pltpu.VMEM((2 * T,), jnp.int32),
            pltpu.VMEM((2 * T,), jnp.float32),
            pltpu.VMEM((PAD,), jnp.int32),
            pltpu.VMEM((PAD,), jnp.float32),
            pltpu.VMEM((RPT // 2, H), jnp.float32),
            pltpu.SemaphoreType.DMA,
        ],
        compiler_params=pltpu.CompilerParams(needs_layout_passes=False),
    )


def _gather_body(h_hbm, pos_hbm, wn_hbm, hs_hbm, rw_hbm,
                 pos_v, wn_v, tok_v, rw_v, buf_v, sem):
    wid = lax.axis_index("s") * 2 + lax.axis_index("c")
    pltpu.sync_copy(pos_hbm, pos_v)
    pltpu.sync_copy(wn_hbm, wn_v)
    iota_init = lax.iota(jnp.int32, L)
    zf = jnp.zeros((L,), jnp.float32)

    def init_body(i, carry):
        sl = pl.ds(i * L, L)
        tok_v[sl] = (iota_init + i * L) & (T - 1)
        rw_v[sl] = zf
        return carry

    lax.fori_loop(0, PAD // L, init_body, 0)
    iota16 = lax.iota(jnp.int32, L)

    def scat_body(i, carry):
        sl = pl.ds(i * L, L)
        idx = pos_v[sl]
        plsc.store_scatter(tok_v, [idx], (iota16 + i * L) & (T - 1))
        plsc.store_scatter(rw_v, [idx], wn_v[sl])
        return carry

    lax.fori_loop(0, 2 * T // L, scat_body, 0)
    base = wid * RPT
    pltpu.sync_copy(rw_v.at[pl.ds(base, RPT)], rw_hbm.at[pl.ds(base, RPT)])
    for half in range(2):
        st = base + half * (RPT // 2)
        pltpu.async_copy(h_hbm.at[tok_v.at[pl.ds(st, RPT // 2)]],
                         buf_v, sem).wait()
        pltpu.sync_copy(buf_v, hs_hbm.at[pl.ds(st, RPT // 2)])


def _ffn_body(be_ref, h_ref, rw_ref, w1_ref, w3_ref, w2_ref, y_ref):
    del be_ref
    h = h_ref[...].astype(jnp.bfloat16)
    w1 = w1_ref[0].astype(jnp.bfloat16)
    w3 = w3_ref[0].astype(jnp.bfloat16)
    w2 = w2_ref[0].astype(jnp.bfloat16)
    nt = (((1,), (1,)), ((), ()))
    f1 = lax.dot_general(h, w1, nt, preferred_element_type=jnp.float32)
    f3 = lax.dot_general(h, w3, nt, preferred_element_type=jnp.float32)
    z = f1 * f3
    x = z / (1.0 + jnp.exp(-z))
    y = lax.dot_general(x.astype(jnp.bfloat16), w2, nt,
                        preferred_element_type=jnp.float32)
    y_ref[...] = y * rw_ref[...]


_ffn = pl.pallas_call(
    _ffn_body,
    grid_spec=pltpu.PrefetchScalarGridSpec(
        num_scalar_prefetch=1,
        grid=(NB,),
        in_specs=[
            pl.BlockSpec((BLK, H), lambda i, be: (i, 0)),
            pl.BlockSpec((BLK, 1), lambda i, be: (i, 0)),
            pl.BlockSpec((1, F, H), lambda i, be: (be[i], 0, 0)),
            pl.BlockSpec((1, F, H), lambda i, be: (be[i], 0, 0)),
            pl.BlockSpec((1, H, F), lambda i, be: (be[i], 0, 0)),
        ],
        out_specs=pl.BlockSpec((BLK, H), lambda i, be: (i, 0)),
    ),
    out_shape=jax.ShapeDtypeStruct((PAD, H), jnp.float32),
)


@functools.cache
def _combine_kernel():
    return pl.kernel(
        _combine_body,
        mesh=_sc_mesh(),
        out_type=jax.ShapeDtypeStruct((T, H), jnp.float32),
        scratch_types=[
            pltpu.VMEM((TPT,), jnp.int32),
            pltpu.VMEM((TPT,), jnp.int32),
            pltpu.VMEM((TPT, H), jnp.float32),
            pltpu.VMEM((TPT, H), jnp.float32),
            pltpu.SemaphoreType.DMA,
        ],
        compiler_params=pltpu.CompilerParams(needs_layout_passes=False),
    )


def _combine_body(y_hbm, pos_hbm, out_hbm, ia_v, ib_v, bufa_v, bufb_v, sem):
    wid = lax.axis_index("s") * 2 + lax.axis_index("c")
    bt = wid * TPT
    pltpu.sync_copy(pos_hbm.at[pl.ds(bt, TPT)], ia_v)
    pltpu.sync_copy(pos_hbm.at[pl.ds(T + bt, TPT)], ib_v)
    pltpu.async_copy(y_hbm.at[ia_v], bufa_v, sem).wait()
    pltpu.async_copy(y_hbm.at[ib_v], bufb_v, sem).wait()

    def tok_body(t, carry):
        for j in range(H // L):
            sl = pl.ds(j * L, L)
            bufa_v[t, sl] = bufa_v[t, sl] + bufb_v[t, sl]
        return carry

    lax.fori_loop(0, TPT, tok_body, 0)
    pltpu.sync_copy(bufa_v, out_hbm.at[pl.ds(bt, TPT)])


def kernel(hidden_states, gate_w, w1, w3, w2):
    h = hidden_states.reshape(T, H)
    gwt = gate_w[GATE].T
    posa, posb, wa, wb, be = _router(h, gwt)
    pos_flat = jnp.concatenate([posa[:, 0], posb[:, 0]])
    wn_flat = jnp.concatenate([wa[:, 0], wb[:, 0]])
    hs, rw = _gather_kernel()(h, pos_flat, wn_flat)
    y = _ffn(be[:, 0], hs, rw.reshape(PAD, 1), w1, w3, w2)
    out = _combine_kernel()(y, pos_flat)
    return out.reshape(1, T, H)


# speedup vs baseline: 1.3022x; 1.0656x over previous
import functools

import jax
import jax.numpy as jnp
from jax import lax
from jax.experimental import pallas as pl
from jax.experimental.pallas import tpu as pltpu
from jax.experimental.pallas import tpu_sc as plsc

T = 2048
H = 768
F = 1280
E = 12
GATE = 4
BLK = 256
NB = T * 2 // BLK + E
PAD = NB * BLK
NTILES = 32
RPT = PAD // NTILES
TPT = T // NTILES
CH = 256
L = 16


def _router_body(h_ref, gwt_ref, posa_ref, posb_ref, wa_ref, wb_ref, be_ref):
    h = h_ref[...]
    logits = jnp.dot(h, gwt_ref[...], preferred_element_type=jnp.float32)
    iota_e = lax.broadcasted_iota(jnp.int32, (T, E), 1)
    m1 = jnp.max(logits, axis=1, keepdims=True)
    a1 = jnp.min(jnp.where(logits >= m1, iota_e, E), axis=1, keepdims=True)
    oh1 = iota_e == a1
    l2 = jnp.where(oh1, jnp.float32(-1e30), logits)
    m2 = jnp.max(l2, axis=1, keepdims=True)
    a2 = jnp.min(jnp.where(l2 >= m2, iota_e, E), axis=1, keepdims=True)
    oh2 = iota_e == a2
    wa = 1.0 / (1.0 + jnp.exp(m2 - m1))
    wa_ref[...] = wa
    wb_ref[...] = 1.0 - wa

    ohc = oh1.astype(jnp.float32) + oh2.astype(jnp.float32)
    r = lax.broadcasted_iota(jnp.int32, (CH, CH), 0)
    c = lax.broadcasted_iota(jnp.int32, (CH, CH), 1)
    stril = (c < r).astype(jnp.float32)
    parts = []
    run = jnp.zeros((1, E), jnp.float32)
    for k in range(T // CH):
        blk = lax.slice(ohc, (k * CH, 0), ((k + 1) * CH, E))
        parts.append(jnp.dot(stril, blk, preferred_element_type=jnp.float32) + run)
        run = run + jnp.sum(blk, axis=0, keepdims=True)
    excl = jnp.concatenate(parts, axis=0)
    counts = run.astype(jnp.int32)
    pc = ((counts + (BLK - 1)) // BLK) * BLK
    rr = lax.broadcasted_iota(jnp.int32, (E, E), 0)
    cc = lax.broadcasted_iota(jnp.int32, (E, E), 1)
    striu = (rr < cc).astype(jnp.float32)
    po = jnp.dot(pc.astype(jnp.float32), striu,
                 preferred_element_type=jnp.float32)
    posf = excl + po
    posa_ref[...] = jnp.sum(jnp.where(oh1, posf, 0.0), axis=1,
                            keepdims=True).astype(jnp.int32)
    posb_ref[...] = jnp.sum(jnp.where(oh2, posf, 0.0), axis=1,
                            keepdims=True).astype(jnp.int32)
    total = jnp.sum(pc)
    sb = jnp.minimum(lax.broadcasted_iota(jnp.int32, (NB, 1), 0) * BLK,
                     total - 1)
    be_ref[...] = jnp.sum((po.astype(jnp.int32) <= sb).astype(jnp.int32),
                          axis=1, keepdims=True) - 1


_router = pl.pallas_call(
    _router_body,
    out_shape=(
        jax.ShapeDtypeStruct((T, 1), jnp.int32),
        jax.ShapeDtypeStruct((T, 1), jnp.int32),
        jax.ShapeDtypeStruct((T, 1), jnp.float32),
        jax.ShapeDtypeStruct((T, 1), jnp.float32),
        jax.ShapeDtypeStruct((NB, 1), jnp.int32),
    ),
)


@functools.cache
def _sc_mesh():
    return plsc.VectorSubcoreMesh(core_axis_name="c", subcore_axis_name="s")


PPT = 2 * T // NTILES


@functools.cache
def _gather_kernel():
    return pl.kernel(
        _gather_body,
        mesh=_sc_mesh(),
        out_type=(
            jax.ShapeDtypeStruct((PAD, H), jnp.float32),
            jax.ShapeDtypeStruct((PAD,), jnp.float32),
        ),
        scratch_types=[
            pltpu.VMEM((2 * T,), jnp.int32),
            pltpu.VMEM((2 * T,), jnp.float32),
            pltpu.VMEM((PAD,), jnp.float32),
            pltpu.VMEM((PPT,), jnp.int32),
            pltpu.VMEM((PPT, H), jnp.float32),
            pltpu.SemaphoreType.DMA,
            pltpu.SemaphoreType.DMA,
        ],
        compiler_params=pltpu.CompilerParams(needs_layout_passes=False),
    )


def _gather_body(h_hbm, pos_hbm, wn_hbm, hs_hbm, rw_hbm,
                 pos_v, wn_v, rw_v, idx_v, buf_v, sem, sem2):
    wid = lax.axis_index("s") * 2 + lax.axis_index("c")
    cp_rows = pltpu.async_copy(
        h_hbm.at[pl.ds((wid & (NTILES // 2 - 1)) * PPT, PPT)], buf_v, sem)
    pltpu.sync_copy(pos_hbm, pos_v)
    pltpu.sync_copy(wn_hbm, wn_v)

    def scat_body(i, carry):
        sl = pl.ds(i * L, L)
        plsc.store_scatter(rw_v, [pos_v[sl]], wn_v[sl])
        return carry

    lax.fori_loop(0, 2 * T // L, scat_body, 0)
    base = wid * RPT
    pltpu.sync_copy(rw_v.at[pl.ds(base, RPT)], rw_hbm.at[pl.ds(base, RPT)])
    for j in range(PPT // L):
        idx_v[pl.ds(j * L, L)] = pos_v[pl.ds(wid * PPT + j * L, L)]
    cp_rows.wait()
    pltpu.async_copy(buf_v, hs_hbm.at[idx_v], sem2).wait()


def _ffn_body(be_ref, h_ref, rw_ref, w1_ref, w3_ref, w2_ref, y_ref):
    del be_ref
    h = h_ref[...].astype(jnp.bfloat16)
    w1 = w1_ref[0].astype(jnp.bfloat16)
    w3 = w3_ref[0].astype(jnp.bfloat16)
    w2 = w2_ref[0].astype(jnp.bfloat16)
    nt = (((1,), (1,)), ((), ()))
    f1 = lax.dot_general(h, w1, nt, preferred_element_type=jnp.float32)
    f3 = lax.dot_general(h, w3, nt, preferred_element_type=jnp.float32)
    z = f1 * f3
    x = z / (1.0 + jnp.exp(-z))
    y = lax.dot_general(x.astype(jnp.bfloat16), w2, nt,
                        preferred_element_type=jnp.float32)
    y_ref[...] = y * rw_ref[...]


_ffn = pl.pallas_call(
    _ffn_body,
    grid_spec=pltpu.PrefetchScalarGridSpec(
        num_scalar_prefetch=1,
        grid=(NB,),
        in_specs=[
            pl.BlockSpec((BLK, H), lambda i, be: (i, 0)),
            pl.BlockSpec((BLK, 1), lambda i, be: (i, 0)),
            pl.BlockSpec((1, F, H), lambda i, be: (be[i], 0, 0)),
            pl.BlockSpec((1, F, H), lambda i, be: (be[i], 0, 0)),
            pl.BlockSpec((1, H, F), lambda i, be: (be[i], 0, 0)),
        ],
        out_specs=pl.BlockSpec((BLK, H), lambda i, be: (i, 0)),
    ),
    out_shape=jax.ShapeDtypeStruct((PAD, H), jnp.float32),
)


@functools.cache
def _combine_kernel():
    return pl.kernel(
        _combine_body,
        mesh=_sc_mesh(),
        out_type=jax.ShapeDtypeStruct((T, H), jnp.float32),
        scratch_types=[
            pltpu.VMEM((TPT,), jnp.int32),
            pltpu.VMEM((TPT,), jnp.int32),
            pltpu.VMEM((TPT, H), jnp.float32),
            pltpu.VMEM((TPT, H), jnp.float32),
            pltpu.SemaphoreType.DMA,
        ],
        compiler_params=pltpu.CompilerParams(needs_layout_passes=False),
    )


def _combine_body(y_hbm, pos_hbm, out_hbm, ia_v, ib_v, bufa_v, bufb_v, sem):
    wid = lax.axis_index("s") * 2 + lax.axis_index("c")
    bt = wid * TPT
    pltpu.sync_copy(pos_hbm.at[pl.ds(bt, TPT)], ia_v)
    pltpu.sync_copy(pos_hbm.at[pl.ds(T + bt, TPT)], ib_v)
    pltpu.async_copy(y_hbm.at[ia_v], bufa_v, sem).wait()
    pltpu.async_copy(y_hbm.at[ib_v], bufb_v, sem).wait()

    def tok_body(t, carry):
        for j in range(H // L):
            sl = pl.ds(j * L, L)
            bufa_v[t, sl] = bufa_v[t, sl] + bufb_v[t, sl]
        return carry

    lax.fori_loop(0, TPT, tok_body, 0)
    pltpu.sync_copy(bufa_v, out_hbm.at[pl.ds(bt, TPT)])


def kernel(hidden_states, gate_w, w1, w3, w2):
    h = hidden_states.reshape(T, H)
    gwt = gate_w[GATE].T
    posa, posb, wa, wb, be = _router(h, gwt)
    pos_flat = jnp.concatenate([posa[:, 0], posb[:, 0]])
    wn_flat = jnp.concatenate([wa[:, 0], wb[:, 0]])
    hs, rw = _gather_kernel()(h, pos_flat, wn_flat)
    y = _ffn(be[:, 0], hs, rw.reshape(PAD, 1), w1, w3, w2)
    out = _combine_kernel()(y, pos_flat)
    return out.reshape(1, T, H)


# speedup vs baseline: 1.3104x; 1.0063x over previous
import functools

import jax
import jax.numpy as jnp
from jax import lax
from jax.experimental import pallas as pl
from jax.experimental.pallas import tpu as pltpu
from jax.experimental.pallas import tpu_sc as plsc

T = 2048
H = 768
F = 1280
E = 12
GATE = 4
BLK = 256
NB = T * 2 // BLK + E
PAD = NB * BLK
NTILES = 32
RPT = PAD // NTILES
TPT = T // NTILES
CH = 256
L = 16


def _router_body(h_ref, gwt_ref, posa_ref, posb_ref, wa_ref, wb_ref, be_ref):
    h = h_ref[...]
    logits = jnp.dot(h, gwt_ref[...], preferred_element_type=jnp.float32)
    iota_e = lax.broadcasted_iota(jnp.int32, (T, E), 1)
    m1 = jnp.max(logits, axis=1, keepdims=True)
    a1 = jnp.min(jnp.where(logits >= m1, iota_e, E), axis=1, keepdims=True)
    oh1 = iota_e == a1
    l2 = jnp.where(oh1, jnp.float32(-1e30), logits)
    m2 = jnp.max(l2, axis=1, keepdims=True)
    a2 = jnp.min(jnp.where(l2 >= m2, iota_e, E), axis=1, keepdims=True)
    oh2 = iota_e == a2
    wa = 1.0 / (1.0 + jnp.exp(m2 - m1))
    wa_ref[...] = wa
    wb_ref[...] = 1.0 - wa

    ohc = oh1.astype(jnp.float32) + oh2.astype(jnp.float32)
    r = lax.broadcasted_iota(jnp.int32, (CH, CH), 0)
    c = lax.broadcasted_iota(jnp.int32, (CH, CH), 1)
    stril = (c < r).astype(jnp.float32)
    parts = []
    run = jnp.zeros((1, E), jnp.float32)
    for k in range(T // CH):
        blk = lax.slice(ohc, (k * CH, 0), ((k + 1) * CH, E))
        parts.append(jnp.dot(stril, blk, preferred_element_type=jnp.float32) + run)
        run = run + jnp.sum(blk, axis=0, keepdims=True)
    excl = jnp.concatenate(parts, axis=0)
    counts = run.astype(jnp.int32)
    pc = ((counts + (BLK - 1)) // BLK) * BLK
    rr = lax.broadcasted_iota(jnp.int32, (E, E), 0)
    cc = lax.broadcasted_iota(jnp.int32, (E, E), 1)
    striu = (rr < cc).astype(jnp.float32)
    po = jnp.dot(pc.astype(jnp.float32), striu,
                 preferred_element_type=jnp.float32)
    posf = excl + po
    posa_ref[...] = jnp.sum(jnp.where(oh1, posf, 0.0), axis=1,
                            keepdims=True).astype(jnp.int32)
    posb_ref[...] = jnp.sum(jnp.where(oh2, posf, 0.0), axis=1,
                            keepdims=True).astype(jnp.int32)
    total = jnp.sum(pc)
    sb = jnp.minimum(lax.broadcasted_iota(jnp.int32, (NB, 1), 0) * BLK,
                     total - 1)
    be_ref[...] = jnp.sum((po.astype(jnp.int32) <= sb).astype(jnp.int32),
                          axis=1, keepdims=True) - 1


_router = pl.pallas_call(
    _router_body,
    out_shape=(
        jax.ShapeDtypeStruct((T, 1), jnp.int32),
        jax.ShapeDtypeStruct((T, 1), jnp.int32),
        jax.ShapeDtypeStruct((T, 1), jnp.float32),
        jax.ShapeDtypeStruct((T, 1), jnp.float32),
        jax.ShapeDtypeStruct((NB, 1), jnp.int32),
    ),
)


@functools.cache
def _sc_mesh():
    return plsc.VectorSubcoreMesh(core_axis_name="c", subcore_axis_name="s")


PPT = 2 * T // NTILES


@functools.cache
def _gather_kernel():
    return pl.kernel(
        _gather_body,
        mesh=_sc_mesh(),
        out_type=(
            jax.ShapeDtypeStruct((PAD, H), jnp.float32),
            jax.ShapeDtypeStruct((PAD,), jnp.float32),
        ),
        scratch_types=[
            pltpu.VMEM((2 * T,), jnp.int32),
            pltpu.VMEM((2 * T,), jnp.float32),
            pltpu.VMEM((PAD,), jnp.float32),
            pltpu.VMEM((PPT,), jnp.int32),
            pltpu.VMEM((PPT, H), jnp.float32),
            pltpu.SemaphoreType.DMA,
            pltpu.SemaphoreType.DMA,
        ],
        compiler_params=pltpu.CompilerParams(needs_layout_passes=False),
    )


def _gather_body(h_hbm, pos_hbm, wn_hbm, hs_hbm, rw_hbm,
                 pos_v, wn_v, rw_v, idx_v, buf_v, sem, sem2):
    wid = lax.axis_index("s") * 2 + lax.axis_index("c")
    cp_rows = pltpu.async_copy(
        h_hbm.at[pl.ds((wid & (NTILES // 2 - 1)) * PPT, PPT)], buf_v, sem)
    pltpu.sync_copy(pos_hbm, pos_v)
    pltpu.sync_copy(wn_hbm, wn_v)

    def scat_body(i, carry):
        sl = pl.ds(i * L, L)
        plsc.store_scatter(rw_v, [pos_v[sl]], wn_v[sl])
        return carry

    lax.fori_loop(0, 2 * T // L, scat_body, 0)
    base = wid * RPT
    pltpu.sync_copy(rw_v.at[pl.ds(base, RPT)], rw_hbm.at[pl.ds(base, RPT)])
    for j in range(PPT // L):
        idx_v[pl.ds(j * L, L)] = pos_v[pl.ds(wid * PPT + j * L, L)]
    cp_rows.wait()
    pltpu.async_copy(buf_v, hs_hbm.at[idx_v], sem2).wait()


def _ffn_body(be_ref, h_ref, rw_ref, w1_ref, w3_ref, w2_ref, y_ref):
    del be_ref
    h = h_ref[...].astype(jnp.bfloat16)
    w1 = w1_ref[0].astype(jnp.bfloat16)
    w3 = w3_ref[0].astype(jnp.bfloat16)
    w2 = w2_ref[0].astype(jnp.bfloat16)
    nt = (((1,), (1,)), ((), ()))
    f1 = lax.dot_general(h, w1, nt, preferred_element_type=jnp.float32)
    f3 = lax.dot_general(h, w3, nt, preferred_element_type=jnp.float32)
    z = f1 * f3
    x = z / (1.0 + jnp.exp(-z))
    y = lax.dot_general(x.astype(jnp.bfloat16), w2, nt,
                        preferred_element_type=jnp.float32)
    y_ref[...] = y * rw_ref[...]


_ffn = pl.pallas_call(
    _ffn_body,
    grid_spec=pltpu.PrefetchScalarGridSpec(
        num_scalar_prefetch=1,
        grid=(NB,),
        in_specs=[
            pl.BlockSpec((BLK, H), lambda i, be: (i, 0)),
            pl.BlockSpec((BLK, 1), lambda i, be: (i, 0)),
            pl.BlockSpec((1, F, H), lambda i, be: (be[i], 0, 0)),
            pl.BlockSpec((1, F, H), lambda i, be: (be[i], 0, 0)),
            pl.BlockSpec((1, H, F), lambda i, be: (be[i], 0, 0)),
        ],
        out_specs=pl.BlockSpec((BLK, H), lambda i, be: (i, 0)),
    ),
    out_shape=jax.ShapeDtypeStruct((PAD, H), jnp.float32),
)


@functools.cache
def _combine_kernel():
    return pl.kernel(
        _combine_body,
        mesh=_sc_mesh(),
        out_type=jax.ShapeDtypeStruct((T, H), jnp.float32),
        scratch_types=[
            pltpu.VMEM((TPT,), jnp.int32),
            pltpu.VMEM((TPT,), jnp.int32),
            pltpu.VMEM((TPT, H), jnp.float32),
            pltpu.VMEM((TPT, H), jnp.float32),
            pltpu.SemaphoreType.DMA,
        ],
        compiler_params=pltpu.CompilerParams(needs_layout_passes=False),
    )


def _combine_body(y_hbm, pos_hbm, out_hbm, ia_v, ib_v, bufa_v, bufb_v, sem):
    wid = lax.axis_index("s") * 2 + lax.axis_index("c")
    bt = wid * TPT
    pltpu.sync_copy(pos_hbm.at[pl.ds(bt, TPT)], ia_v)
    pltpu.sync_copy(pos_hbm.at[pl.ds(T + bt, TPT)], ib_v)
    cpa = pltpu.async_copy(y_hbm.at[ia_v], bufa_v, sem)
    cpb = pltpu.async_copy(y_hbm.at[ib_v], bufb_v, sem)
    cpa.wait()
    cpb.wait()

    def tok_body(t, carry):
        for j in range(H // L):
            sl = pl.ds(j * L, L)
            bufa_v[t, sl] = bufa_v[t, sl] + bufb_v[t, sl]
        return carry

    lax.fori_loop(0, TPT, tok_body, 0)
    pltpu.sync_copy(bufa_v, out_hbm.at[pl.ds(bt, TPT)])


def kernel(hidden_states, gate_w, w1, w3, w2):
    h = hidden_states.reshape(T, H)
    gwt = gate_w[GATE].T
    posa, posb, wa, wb, be = _router(h, gwt)
    pos_flat = jnp.concatenate([posa[:, 0], posb[:, 0]])
    wn_flat = jnp.concatenate([wa[:, 0], wb[:, 0]])
    hs, rw = _gather_kernel()(h, pos_flat, wn_flat)
    y = _ffn(be[:, 0], hs, rw.reshape(PAD, 1), w1, w3, w2)
    out = _combine_kernel()(y, pos_flat)
    return out.reshape(1, T, H)


# speedup vs baseline: 1.4166x; 1.0810x over previous
import functools

import jax
import jax.numpy as jnp
from jax import lax
from jax.experimental import pallas as pl
from jax.experimental.pallas import tpu as pltpu
from jax.experimental.pallas import tpu_sc as plsc

T = 2048
H = 768
F = 1280
E = 12
GATE = 4
BLK = 256
NB = T * 2 // BLK + E
PAD = NB * BLK
NTILES = 32
TPT = T // NTILES
PPT = 2 * T // NTILES
CH = 256
L = 16


def _router_body(h_ref, gw_ref, posa_ref, posb_ref, wa_ref, wb_ref, be_ref):
    h = h_ref[...]
    nt = (((1,), (1,)), ((), ()))
    logits = lax.dot_general(h, gw_ref[...], nt,
                             preferred_element_type=jnp.float32)
    iota_e = lax.broadcasted_iota(jnp.int32, (T, E), 1)
    m1 = jnp.max(logits, axis=1, keepdims=True)
    a1 = jnp.min(jnp.where(logits >= m1, iota_e, E), axis=1, keepdims=True)
    oh1 = iota_e == a1
    l2 = jnp.where(oh1, jnp.float32(-1e30), logits)
    m2 = jnp.max(l2, axis=1, keepdims=True)
    a2 = jnp.min(jnp.where(l2 >= m2, iota_e, E), axis=1, keepdims=True)
    oh2 = iota_e == a2
    wa = 1.0 / (1.0 + jnp.exp(m2 - m1))
    wa_ref[...] = wa.reshape(T)
    wb_ref[...] = (1.0 - wa).reshape(T)

    ohc = oh1.astype(jnp.float32) + oh2.astype(jnp.float32)
    r = lax.broadcasted_iota(jnp.int32, (CH, CH), 0)
    c = lax.broadcasted_iota(jnp.int32, (CH, CH), 1)
    stril = (c < r).astype(jnp.float32)
    parts = []
    run = jnp.zeros((1, E), jnp.float32)
    for k in range(T // CH):
        blk = lax.slice(ohc, (k * CH, 0), ((k + 1) * CH, E))
        parts.append(jnp.dot(stril, blk, preferred_element_type=jnp.float32) + run)
        run = run + jnp.sum(blk, axis=0, keepdims=True)
    excl = jnp.concatenate(parts, axis=0)
    counts = run.astype(jnp.int32)
    pc = ((counts + (BLK - 1)) // BLK) * BLK
    rr = lax.broadcasted_iota(jnp.int32, (E, E), 0)
    cc = lax.broadcasted_iota(jnp.int32, (E, E), 1)
    striu = (rr < cc).astype(jnp.float32)
    po = jnp.dot(pc.astype(jnp.float32), striu,
                 preferred_element_type=jnp.float32)
    posf = excl + po
    posa_ref[...] = jnp.sum(jnp.where(oh1, posf, 0.0), axis=1).astype(jnp.int32)
    posb_ref[...] = jnp.sum(jnp.where(oh2, posf, 0.0), axis=1).astype(jnp.int32)
    total = jnp.sum(pc)
    sb = jnp.minimum(lax.broadcasted_iota(jnp.int32, (NB, 1), 0) * BLK,
                     total - 1)
    be_ref[...] = (jnp.sum((po.astype(jnp.int32) <= sb).astype(jnp.int32),
                           axis=1) - 1)


_router = pl.pallas_call(
    _router_body,
    out_shape=(
        jax.ShapeDtypeStruct((T,), jnp.int32),
        jax.ShapeDtypeStruct((T,), jnp.int32),
        jax.ShapeDtypeStruct((T,), jnp.float32),
        jax.ShapeDtypeStruct((T,), jnp.float32),
        jax.ShapeDtypeStruct((NB,), jnp.int32),
    ),
)


@functools.cache
def _sc_mesh():
    return plsc.VectorSubcoreMesh(core_axis_name="c", subcore_axis_name="s")


@functools.cache
def _scatter_kernel():
    return pl.kernel(
        _scatter_body,
        mesh=_sc_mesh(),
        out_type=jax.ShapeDtypeStruct((PAD, H), jnp.float32),
        scratch_types=[
            pltpu.VMEM((PPT,), jnp.int32),
            pltpu.VMEM((PPT, H), jnp.float32),
            pltpu.SemaphoreType.DMA,
            pltpu.SemaphoreType.DMA,
        ],
        compiler_params=pltpu.CompilerParams(needs_layout_passes=False),
    )


def _scatter_body(h_hbm, posa_hbm, posb_hbm, hs_hbm, idx_v, buf_v, sem, sem2):
    wid = lax.axis_index("s") * 2 + lax.axis_index("c")
    half = wid & (NTILES // 2 - 1)
    cp_rows = pltpu.async_copy(h_hbm.at[pl.ds(half * PPT, PPT)], buf_v, sem)
    @pl.when(wid < NTILES // 2)
    def _():
        pltpu.sync_copy(posa_hbm.at[pl.ds(half * PPT, PPT)], idx_v)

    @pl.when(wid >= NTILES // 2)
    def _():
        pltpu.sync_copy(posb_hbm.at[pl.ds(half * PPT, PPT)], idx_v)

    cp_rows.wait()
    pltpu.async_copy(buf_v, hs_hbm.at[idx_v], sem2).wait()


def _ffn_body(be_ref, h_ref, w1_ref, w3_ref, w2_ref, y_ref):
    del be_ref
    h = h_ref[...].astype(jnp.bfloat16)
    w1 = w1_ref[0].astype(jnp.bfloat16)
    w3 = w3_ref[0].astype(jnp.bfloat16)
    w2 = w2_ref[0].astype(jnp.bfloat16)
    nt = (((1,), (1,)), ((), ()))
    f1 = lax.dot_general(h, w1, nt, preferred_element_type=jnp.float32)
    f3 = lax.dot_general(h, w3, nt, preferred_element_type=jnp.float32)
    z = f1 * f3
    x = z / (1.0 + jnp.exp(-z))
    y_ref[...] = lax.dot_general(x.astype(jnp.bfloat16), w2, nt,
                                 preferred_element_type=jnp.float32)


_ffn = pl.pallas_call(
    _ffn_body,
    grid_spec=pltpu.PrefetchScalarGridSpec(
        num_scalar_prefetch=1,
        grid=(NB,),
        in_specs=[
            pl.BlockSpec((BLK, H), lambda i, be: (i, 0)),
            pl.BlockSpec((1, F, H), lambda i, be: (be[i], 0, 0)),
            pl.BlockSpec((1, F, H), lambda i, be: (be[i], 0, 0)),
            pl.BlockSpec((1, H, F), lambda i, be: (be[i], 0, 0)),
        ],
        out_specs=pl.BlockSpec((BLK, H), lambda i, be: (i, 0)),
    ),
    out_shape=jax.ShapeDtypeStruct((PAD, H), jnp.float32),
)


@functools.cache
def _combine_kernel():
    return pl.kernel(
        _combine_body,
        mesh=_sc_mesh(),
        out_type=jax.ShapeDtypeStruct((T, H), jnp.float32),
        scratch_types=[
            pltpu.VMEM((TPT,), jnp.int32),
            pltpu.VMEM((TPT,), jnp.int32),
            pltpu.VMEM((TPT,), jnp.float32),
            pltpu.VMEM((TPT,), jnp.float32),
            pltpu.VMEM((TPT, H), jnp.float32),
            pltpu.VMEM((TPT, H), jnp.float32),
            pltpu.SemaphoreType.DMA,
        ],
        compiler_params=pltpu.CompilerParams(needs_layout_passes=False),
    )


def _combine_body(y_hbm, posa_hbm, posb_hbm, wa_hbm, wb_hbm, out_hbm,
                  ia_v, ib_v, wa_v, wb_v, bufa_v, bufb_v, sem):
    wid = lax.axis_index("s") * 2 + lax.axis_index("c")
    bt = wid * TPT
    pltpu.sync_copy(posa_hbm.at[pl.ds(bt, TPT)], ia_v)
    pltpu.sync_copy(posb_hbm.at[pl.ds(bt, TPT)], ib_v)
    cpa = pltpu.async_copy(y_hbm.at[ia_v], bufa_v, sem)
    cpb = pltpu.async_copy(y_hbm.at[ib_v], bufb_v, sem)
    pltpu.sync_copy(wa_hbm.at[pl.ds(bt, TPT)], wa_v)
    pltpu.sync_copy(wb_hbm.at[pl.ds(bt, TPT)], wb_v)
    cpa.wait()
    cpb.wait()

    def tok_body(t, carry):
        tsplat = jnp.full((L,), t, jnp.int32)
        wav = plsc.load_gather(wa_v, [tsplat])
        wbv = plsc.load_gather(wb_v, [tsplat])
        for j in range(H // L):
            sl = pl.ds(j * L, L)
            bufa_v[t, sl] = bufa_v[t, sl] * wav + bufb_v[t, sl] * wbv
        return carry

    lax.fori_loop(0, TPT, tok_body, 0)
    pltpu.sync_copy(bufa_v, out_hbm.at[pl.ds(bt, TPT)])


def kernel(hidden_states, gate_w, w1, w3, w2):
    h = hidden_states.reshape(T, H)
    posa, posb, wa, wb, be = _router(h, gate_w[GATE])
    hs = _scatter_kernel()(h, posa, posb)
    y = _ffn(be, hs, w1, w3, w2)
    out = _combine_kernel()(y, posa, posb, wa, wb)
    return out.reshape(1, T, H)


# speedup vs baseline: 1.4321x; 1.0109x over previous
import functools

import jax
import jax.numpy as jnp
from jax import lax
from jax.experimental import pallas as pl
from jax.experimental.pallas import tpu as pltpu
from jax.experimental.pallas import tpu_sc as plsc

T = 2048
H = 768
F = 1280
E = 12
GATE = 4
BLK = 256
NB = T * 2 // BLK + E
PAD = NB * BLK
NTILES = 32
TPT = T // NTILES
PPT = 2 * T // NTILES
CH = 256
L = 16


def _router_body(h_ref, gw_ref, posa_ref, posb_ref, wa_ref, wb_ref, be_ref):
    h = h_ref[...]
    nt = (((1,), (1,)), ((), ()))
    logits = lax.dot_general(h, gw_ref[0], nt,
                             preferred_element_type=jnp.float32)
    iota_e = lax.broadcasted_iota(jnp.int32, (T, E), 1)
    m1 = jnp.max(logits, axis=1, keepdims=True)
    a1 = jnp.min(jnp.where(logits >= m1, iota_e, E), axis=1, keepdims=True)
    oh1 = iota_e == a1
    l2 = jnp.where(oh1, jnp.float32(-1e30), logits)
    m2 = jnp.max(l2, axis=1, keepdims=True)
    a2 = jnp.min(jnp.where(l2 >= m2, iota_e, E), axis=1, keepdims=True)
    oh2 = iota_e == a2
    wa = 1.0 / (1.0 + jnp.exp(m2 - m1))
    wa_ref[...] = wa.reshape(T)
    wb_ref[...] = (1.0 - wa).reshape(T)

    ohc = oh1.astype(jnp.float32) + oh2.astype(jnp.float32)
    r = lax.broadcasted_iota(jnp.int32, (CH, CH), 0)
    c = lax.broadcasted_iota(jnp.int32, (CH, CH), 1)
    stril = (c < r).astype(jnp.float32)
    parts = []
    run = jnp.zeros((1, E), jnp.float32)
    for k in range(T // CH):
        blk = lax.slice(ohc, (k * CH, 0), ((k + 1) * CH, E))
        parts.append(jnp.dot(stril, blk, preferred_element_type=jnp.float32) + run)
        run = run + jnp.sum(blk, axis=0, keepdims=True)
    excl = jnp.concatenate(parts, axis=0)
    counts = run.astype(jnp.int32)
    pc = ((counts + (BLK - 1)) // BLK) * BLK
    rr = lax.broadcasted_iota(jnp.int32, (E, E), 0)
    cc = lax.broadcasted_iota(jnp.int32, (E, E), 1)
    striu = (rr < cc).astype(jnp.float32)
    po = jnp.dot(pc.astype(jnp.float32), striu,
                 preferred_element_type=jnp.float32)
    posf = excl + po
    posa_ref[...] = jnp.sum(jnp.where(oh1, posf, 0.0), axis=1).astype(jnp.int32)
    posb_ref[...] = jnp.sum(jnp.where(oh2, posf, 0.0), axis=1).astype(jnp.int32)
    total = jnp.sum(pc)
    sb = jnp.minimum(lax.broadcasted_iota(jnp.int32, (NB, 1), 0) * BLK,
                     total - 1)
    be_ref[...] = (jnp.sum((po.astype(jnp.int32) <= sb).astype(jnp.int32),
                           axis=1) - 1)


_router = pl.pallas_call(
    _router_body,
    grid=(1,),
    in_specs=[
        pl.BlockSpec((T, H), lambda i: (0, 0)),
        pl.BlockSpec((1, E, H), lambda i: (GATE, 0, 0)),
    ],
    out_specs=(
        pl.BlockSpec((T,), lambda i: (0,)),
        pl.BlockSpec((T,), lambda i: (0,)),
        pl.BlockSpec((T,), lambda i: (0,)),
        pl.BlockSpec((T,), lambda i: (0,)),
        pl.BlockSpec((NB,), lambda i: (0,)),
    ),
    out_shape=(
        jax.ShapeDtypeStruct((T,), jnp.int32),
        jax.ShapeDtypeStruct((T,), jnp.int32),
        jax.ShapeDtypeStruct((T,), jnp.float32),
        jax.ShapeDtypeStruct((T,), jnp.float32),
        jax.ShapeDtypeStruct((NB,), jnp.int32),
    ),
)


@functools.cache
def _sc_mesh():
    return plsc.VectorSubcoreMesh(core_axis_name="c", subcore_axis_name="s")


@functools.cache
def _scatter_kernel():
    return pl.kernel(
        _scatter_body,
        mesh=_sc_mesh(),
        out_type=jax.ShapeDtypeStruct((PAD, H), jnp.float32),
        scratch_types=[
            pltpu.VMEM((PPT,), jnp.int32),
            pltpu.VMEM((PPT, H), jnp.float32),
            pltpu.SemaphoreType.DMA,
            pltpu.SemaphoreType.DMA,
        ],
        compiler_params=pltpu.CompilerParams(needs_layout_passes=False),
    )


def _scatter_body(h_hbm, posa_hbm, posb_hbm, hs_hbm, idx_v, buf_v, sem, sem2):
    wid = lax.axis_index("s") * 2 + lax.axis_index("c")
    half = wid & (NTILES // 2 - 1)
    cp_rows = pltpu.async_copy(h_hbm.at[pl.ds(half * PPT, PPT)], buf_v, sem)
    @pl.when(wid < NTILES // 2)
    def _():
        pltpu.sync_copy(posa_hbm.at[pl.ds(half * PPT, PPT)], idx_v)

    @pl.when(wid >= NTILES // 2)
    def _():
        pltpu.sync_copy(posb_hbm.at[pl.ds(half * PPT, PPT)], idx_v)

    cp_rows.wait()
    pltpu.async_copy(buf_v, hs_hbm.at[idx_v], sem2).wait()


def _ffn_body(be_ref, h_ref, w1_ref, w3_ref, w2_ref, y_ref):
    del be_ref
    h = h_ref[...].astype(jnp.bfloat16)
    w1 = w1_ref[0].astype(jnp.bfloat16)
    w3 = w3_ref[0].astype(jnp.bfloat16)
    w2 = w2_ref[0].astype(jnp.bfloat16)
    nt = (((1,), (1,)), ((), ()))
    f1 = lax.dot_general(h, w1, nt, preferred_element_type=jnp.float32)
    f3 = lax.dot_general(h, w3, nt, preferred_element_type=jnp.float32)
    z = f1 * f3
    x = z / (1.0 + jnp.exp(-z))
    y_ref[...] = lax.dot_general(x.astype(jnp.bfloat16), w2, nt,
                                 preferred_element_type=jnp.float32)


_ffn = pl.pallas_call(
    _ffn_body,
    grid_spec=pltpu.PrefetchScalarGridSpec(
        num_scalar_prefetch=1,
        grid=(NB,),
        in_specs=[
            pl.BlockSpec((BLK, H), lambda i, be: (i, 0)),
            pl.BlockSpec((1, F, H), lambda i, be: (be[i], 0, 0)),
            pl.BlockSpec((1, F, H), lambda i, be: (be[i], 0, 0)),
            pl.BlockSpec((1, H, F), lambda i, be: (be[i], 0, 0)),
        ],
        out_specs=pl.BlockSpec((BLK, H), lambda i, be: (i, 0)),
    ),
    out_shape=jax.ShapeDtypeStruct((PAD, H), jnp.float32),
)


@functools.cache
def _combine_kernel():
    return pl.kernel(
        _combine_body,
        mesh=_sc_mesh(),
        out_type=jax.ShapeDtypeStruct((T, H), jnp.float32),
        scratch_types=[
            pltpu.VMEM((TPT,), jnp.int32),
            pltpu.VMEM((TPT,), jnp.int32),
            pltpu.VMEM((TPT,), jnp.float32),
            pltpu.VMEM((TPT,), jnp.float32),
            pltpu.VMEM((TPT, H), jnp.float32),
            pltpu.VMEM((TPT, H), jnp.float32),
            pltpu.SemaphoreType.DMA,
        ],
        compiler_params=pltpu.CompilerParams(needs_layout_passes=False),
    )


def _combine_body(y_hbm, posa_hbm, posb_hbm, wa_hbm, wb_hbm, out_hbm,
                  ia_v, ib_v, wa_v, wb_v, bufa_v, bufb_v, sem):
    wid = lax.axis_index("s") * 2 + lax.axis_index("c")
    bt = wid * TPT
    pltpu.sync_copy(posa_hbm.at[pl.ds(bt, TPT)], ia_v)
    pltpu.sync_copy(posb_hbm.at[pl.ds(bt, TPT)], ib_v)
    cpa = pltpu.async_copy(y_hbm.at[ia_v], bufa_v, sem)
    cpb = pltpu.async_copy(y_hbm.at[ib_v], bufb_v, sem)
    pltpu.sync_copy(wa_hbm.at[pl.ds(bt, TPT)], wa_v)
    pltpu.sync_copy(wb_hbm.at[pl.ds(bt, TPT)], wb_v)
    cpa.wait()
    cpb.wait()

    def tok_body(t, carry):
        tsplat = jnp.full((L,), t, jnp.int32)
        wav = plsc.load_gather(wa_v, [tsplat])
        wbv = plsc.load_gather(wb_v, [tsplat])
        for j in range(H // L):
            sl = pl.ds(j * L, L)
            bufa_v[t, sl] = bufa_v[t, sl] * wav + bufb_v[t, sl] * wbv
        return carry

    lax.fori_loop(0, TPT, tok_body, 0)
    pltpu.sync_copy(bufa_v, out_hbm.at[pl.ds(bt, TPT)])


def kernel(hidden_states, gate_w, w1, w3, w2):
    h = hidden_states.reshape(T, H)
    posa, posb, wa, wb, be = _router(h, gate_w)
    hs = _scatter_kernel()(h, posa, posb)
    y = _ffn(be, hs, w1, w3, w2)
    out = _combine_kernel()(y, posa, posb, wa, wb)
    return out.reshape(1, T, H)
